# baseline (speedup 1.0000x reference)
import functools

import jax
import jax.numpy as jnp
from jax import lax
from jax.experimental import pallas as pl
from jax.experimental.pallas import tpu as pltpu
from jax.experimental.pallas import tpu_sc as plsc

N_NODES = 10000
NODE_DIM = 128
EDGE_DIM = 16
HIDDEN_DIM = 128
N_EDGES = 320000

NC = 2
NS = 16
NW = NC * NS
EPW = N_EDGES // NW
CH = 80
K = 5
OUTER = EPW // (CH * K)
ROWS_PER_TILE = N_NODES // NS

HALF = NODE_DIM // 2
KG = 8
OUTER_G = EPW // (KG * CH)
GTAIL = (EPW - OUTER_G * KG * CH) // CH
STRIPE = 1000


@functools.cache
def _make_sc_gather():
    mesh = plsc.VectorSubcoreMesh(core_axis_name="c", subcore_axis_name="s")
    return pl.kernel(
        _sc_gather_body,
        mesh=mesh,
        out_type=jax.ShapeDtypeStruct((N_EDGES, NODE_DIM), jnp.float32),
        scratch_types=[
            pltpu.VMEM((KG * CH,), jnp.int32),
            pltpu.VMEM((KG * CH, NODE_DIM), jnp.float32),
            pltpu.SemaphoreType.DMA,
        ],
    )


def _sc_gather_body(x_hbm, src_hbm, out_hbm, idx_v, rows_v, sem):
    cid = lax.axis_index("c")
    sid = lax.axis_index("s")
    wid = sid * NC + cid

    def step(base, k):
        pltpu.sync_copy(src_hbm.at[pl.ds(base, k * CH)], idx_v.at[pl.ds(0, k * CH)])
        cps = [
            pltpu.async_copy(
                x_hbm.at[idx_v.at[pl.ds(t * CH, CH)]],
                rows_v.at[pl.ds(t * CH, CH)],
                sem,
            )
            for t in range(k)
        ]
        for cp in cps:
            cp.wait()
        pltpu.sync_copy(rows_v.at[pl.ds(0, k * CH)], out_hbm.at[pl.ds(base, k * CH)])

    def body(j, carry):
        step(wid * EPW + j * (KG * CH), KG)
        return carry

    lax.fori_loop(0, OUTER_G, body, 0)
    step(wid * EPW + OUTER_G * (KG * CH), GTAIL)


KS = 4
OUTER_S = EPW // (KS * CH)
TAIL_S = EPW - OUTER_S * KS * CH


@functools.cache
def _make_sc_scatter():
    mesh = plsc.VectorSubcoreMesh(core_axis_name="c", subcore_axis_name="s")
    return pl.kernel(
        _sc_scatter_body,
        mesh=mesh,
        out_type=jax.ShapeDtypeStruct((NC * N_NODES, HIDDEN_DIM), jnp.float32),
        scratch_types=[
            [pltpu.VMEM((CH,), jnp.int32) for _ in range(KS)],
            pltpu.VMEM((KS * CH, HIDDEN_DIM), jnp.float32),
            pltpu.VMEM_SHARED((N_NODES, HIDDEN_DIM), jnp.float32),
            pltpu.SemaphoreType.DMA,
        ],
    )


def _sc_scatter_body(
    msg_hbm, dst_hbm, zero_hbm, out_hbm, idx_vs, rows_v, agg_sh, isem
):
    cid = lax.axis_index("c")
    sid = lax.axis_index("s")
    wid = sid * NC + cid
    @pl.when(sid < N_NODES // STRIPE)
    def _():
        pltpu.sync_copy(
            zero_hbm.at[pl.ds(sid * STRIPE, STRIPE)],
            agg_sh.at[pl.ds(sid * STRIPE, STRIPE)],
        )

    plsc.subcore_barrier()

    def step(base, k):
        icps = [
            pltpu.async_copy(
                dst_hbm.at[pl.ds(base + t * CH, CH)], idx_vs[t], isem
            )
            for t in range(k)
        ]
        pltpu.sync_copy(msg_hbm.at[pl.ds(base, k * CH)], rows_v.at[pl.ds(0, k * CH)])
        for cp in icps:
            cp.wait()
        for t in range(k):
            pltpu.sync_copy(
                rows_v.at[pl.ds(t * CH, CH)], agg_sh.at[idx_vs[t]], add=True
            )

    def body(j, carry):
        step(wid * EPW + j * (KS * CH), KS)
        return carry

    lax.fori_loop(0, OUTER_S, body, 0)
    step(wid * EPW + OUTER_S * (KS * CH), TAIL_S // CH)
    plsc.subcore_barrier()

    @pl.when(sid < N_NODES // STRIPE)
    def _():
        pltpu.sync_copy(
            agg_sh.at[pl.ds(sid * STRIPE, STRIPE)],
            out_hbm.at[pl.ds(cid * N_NODES + sid * STRIPE, STRIPE)],
        )


def _mlp_body(gx_ref, ea_ref, w1x_ref, w1e_ref, b1_ref, w2_ref, b2_ref, out_ref):
    gx = gx_ref[...].astype(jnp.bfloat16)
    ea = ea_ref[...].astype(jnp.bfloat16)
    h = jnp.dot(gx, w1x_ref[...], preferred_element_type=jnp.float32)
    h = h + jnp.dot(ea, w1e_ref[...], preferred_element_type=jnp.float32)
    h = jnp.maximum(h + b1_ref[...], 0.0).astype(jnp.bfloat16)
    out_ref[...] = (
        jnp.dot(h, w2_ref[...], preferred_element_type=jnp.float32) + b2_ref[...]
    )


def _gru_body(a0_ref, a1_ref, x_ref, wih_ref, whh_ref, bih_ref, bhh_ref, out_ref):
    agg = a0_ref[...] + a1_ref[...]
    x = x_ref[...]
    gi = jnp.dot(agg, wih_ref[...], preferred_element_type=jnp.float32) + bih_ref[...]
    gh = jnp.dot(x, whh_ref[...], preferred_element_type=jnp.float32) + bhh_ref[...]
    i_r = gi[:, :NODE_DIM]
    i_z = gi[:, NODE_DIM : 2 * NODE_DIM]
    i_n = gi[:, 2 * NODE_DIM :]
    h_r = gh[:, :NODE_DIM]
    h_z = gh[:, NODE_DIM : 2 * NODE_DIM]
    h_n = gh[:, 2 * NODE_DIM :]
    r = jax.nn.sigmoid(i_r + h_r)
    z = jax.nn.sigmoid(i_z + h_z)
    n = jnp.tanh(i_n + r * h_n)
    out_ref[...] = (1.0 - z) * n + z * x


BE = 4000
BN = 2000


def kernel(x, edge_index, edge_attr, W1, b1, W2, b2, W_ih, b_ih, W_hh, b_hh):
    src = edge_index[0].astype(jnp.int32)
    dst = edge_index[1].astype(jnp.int32)

    gathered = _make_sc_gather()(x, src)

    w1x_t = W1[:, :NODE_DIM].T.astype(jnp.bfloat16)
    w1e_t = W1[:, NODE_DIM:].T.astype(jnp.bfloat16)
    messages = pl.pallas_call(
        _mlp_body,
        grid=(N_EDGES // BE,),
        in_specs=[
            pl.BlockSpec((BE, NODE_DIM), lambda i: (i, 0)),
            pl.BlockSpec((BE, EDGE_DIM), lambda i: (i, 0)),
            pl.BlockSpec((NODE_DIM, HIDDEN_DIM), lambda i: (0, 0)),
            pl.BlockSpec((EDGE_DIM, HIDDEN_DIM), lambda i: (0, 0)),
            pl.BlockSpec((1, HIDDEN_DIM), lambda i: (0, 0)),
            pl.BlockSpec((HIDDEN_DIM, HIDDEN_DIM), lambda i: (0, 0)),
            pl.BlockSpec((1, HIDDEN_DIM), lambda i: (0, 0)),
        ],
        out_specs=pl.BlockSpec((BE, HIDDEN_DIM), lambda i: (i, 0)),
        out_shape=jax.ShapeDtypeStruct((N_EDGES, HIDDEN_DIM), jnp.float32),
    )(
        gathered,
        edge_attr,
        w1x_t,
        w1e_t,
        b1.reshape(1, HIDDEN_DIM),
        W2.T.astype(jnp.bfloat16),
        b2.reshape(1, HIDDEN_DIM),
    )

    zero = jnp.zeros((N_NODES, HIDDEN_DIM), jnp.float32)
    agg2 = _make_sc_scatter()(messages, dst, zero)
    a0 = agg2[:N_NODES]
    a1 = agg2[N_NODES:]

    x_new = pl.pallas_call(
        _gru_body,
        grid=(N_NODES // BN,),
        in_specs=[
            pl.BlockSpec((BN, HIDDEN_DIM), lambda i: (i, 0)),
            pl.BlockSpec((BN, HIDDEN_DIM), lambda i: (i, 0)),
            pl.BlockSpec((BN, NODE_DIM), lambda i: (i, 0)),
            pl.BlockSpec((HIDDEN_DIM, 3 * NODE_DIM), lambda i: (0, 0)),
            pl.BlockSpec((NODE_DIM, 3 * NODE_DIM), lambda i: (0, 0)),
            pl.BlockSpec((1, 3 * NODE_DIM), lambda i: (0, 0)),
            pl.BlockSpec((1, 3 * NODE_DIM), lambda i: (0, 0)),
        ],
        out_specs=pl.BlockSpec((BN, NODE_DIM), lambda i: (i, 0)),
        out_shape=jax.ShapeDtypeStruct((N_NODES, NODE_DIM), jnp.float32),
    )(
        a0,
        a1,
        x,
        W_ih.T,
        W_hh.T,
        b_ih.reshape(1, 3 * NODE_DIM),
        b_hh.reshape(1, 3 * NODE_DIM),
    )
    return x_new


# speedup vs baseline: 1.2095x; 1.2095x over previous
import functools

import jax
import jax.numpy as jnp
from jax import lax
from jax.experimental import pallas as pl
from jax.experimental.pallas import tpu as pltpu
from jax.experimental.pallas import tpu_sc as plsc

N_NODES = 10000
NODE_DIM = 128
EDGE_DIM = 16
HIDDEN_DIM = 128
N_EDGES = 320000

NC = 2
NS = 16
NW = NC * NS
EPW = N_EDGES // NW
CH = 80
K = 5
OUTER = EPW // (CH * K)
ROWS_PER_TILE = N_NODES // NS

HALF = NODE_DIM // 2
KG = 8
OUTER_G = EPW // (KG * CH)
GTAIL = (EPW - OUTER_G * KG * CH) // CH
STRIPE = 1000


@functools.cache
def _make_sc_gather():
    mesh = plsc.VectorSubcoreMesh(core_axis_name="c", subcore_axis_name="s")
    return pl.kernel(
        _sc_gather_body,
        mesh=mesh,
        out_type=jax.ShapeDtypeStruct((N_EDGES, NODE_DIM), jnp.float32),
        scratch_types=[
            pltpu.VMEM((KG * CH,), jnp.int32),
            pltpu.VMEM((KG * CH, NODE_DIM), jnp.float32),
            pltpu.SemaphoreType.DMA,
        ],
    )


def _sc_gather_body(x_hbm, src_hbm, out_hbm, idx_v, rows_v, sem):
    cid = lax.axis_index("c")
    sid = lax.axis_index("s")
    wid = sid * NC + cid

    def step(base, k):
        pltpu.sync_copy(src_hbm.at[pl.ds(base, k * CH)], idx_v.at[pl.ds(0, k * CH)])
        cps = [
            pltpu.async_copy(
                x_hbm.at[idx_v.at[pl.ds(t * CH, CH)]],
                rows_v.at[pl.ds(t * CH, CH)],
                sem,
            )
            for t in range(k)
        ]
        for cp in cps:
            cp.wait()
        pltpu.sync_copy(rows_v.at[pl.ds(0, k * CH)], out_hbm.at[pl.ds(base, k * CH)])

    def body(j, carry):
        step(wid * EPW + j * (KG * CH), KG)
        return carry

    lax.fori_loop(0, OUTER_G, body, 0)
    step(wid * EPW + OUTER_G * (KG * CH), GTAIL)


KS = 4
OUTER_S = EPW // (KS * CH)
TAIL_S = EPW - OUTER_S * KS * CH


@functools.cache
def _make_sc_scatter():
    mesh = plsc.VectorSubcoreMesh(core_axis_name="c", subcore_axis_name="s")
    return pl.kernel(
        _sc_scatter_body,
        mesh=mesh,
        out_type=jax.ShapeDtypeStruct((NC * N_NODES, HIDDEN_DIM), jnp.float32),
        scratch_types=[
            [pltpu.VMEM((CH,), jnp.int32) for _ in range(KS)],
            pltpu.VMEM((KS * CH, HIDDEN_DIM), jnp.float32),
            pltpu.VMEM_SHARED((N_NODES, HIDDEN_DIM), jnp.float32),
            pltpu.SemaphoreType.DMA,
        ],
    )


def _sc_scatter_body(
    msg_hbm, dst_hbm, zero_hbm, out_hbm, idx_vs, rows_v, agg_sh, isem
):
    cid = lax.axis_index("c")
    sid = lax.axis_index("s")
    wid = sid * NC + cid
    @pl.when(sid < N_NODES // STRIPE)
    def _():
        pltpu.sync_copy(
            zero_hbm.at[pl.ds(sid * STRIPE, STRIPE)],
            agg_sh.at[pl.ds(sid * STRIPE, STRIPE)],
        )

    plsc.subcore_barrier()

    def step(base, k):
        icps = [
            pltpu.async_copy(
                dst_hbm.at[pl.ds(base + t * CH, CH)], idx_vs[t], isem
            )
            for t in range(k)
        ]
        pltpu.sync_copy(msg_hbm.at[pl.ds(base, k * CH)], rows_v.at[pl.ds(0, k * CH)])
        for cp in icps:
            cp.wait()
        for t in range(k):
            pltpu.sync_copy(
                rows_v.at[pl.ds(t * CH, CH)], agg_sh.at[idx_vs[t]], add=True
            )

    def body(j, carry):
        step(wid * EPW + j * (KS * CH), KS)
        return carry

    lax.fori_loop(0, OUTER_S, body, 0)
    step(wid * EPW + OUTER_S * (KS * CH), TAIL_S // CH)
    plsc.subcore_barrier()

    @pl.when(sid < N_NODES // STRIPE)
    def _():
        pltpu.sync_copy(
            agg_sh.at[pl.ds(sid * STRIPE, STRIPE)],
            out_hbm.at[pl.ds(cid * N_NODES + sid * STRIPE, STRIPE)],
        )


def _mlp_body(gx_ref, ea_ref, w1x_ref, w1e_ref, b1_ref, w2_ref, b2_ref, out_ref):
    gx = gx_ref[...].astype(jnp.bfloat16)
    ea_t = ea_ref[...].astype(jnp.bfloat16)
    h = jnp.dot(gx, w1x_ref[...], preferred_element_type=jnp.float32)
    h = h + lax.dot_general(
        ea_t,
        w1e_ref[...],
        dimension_numbers=(((0,), (0,)), ((), ())),
        preferred_element_type=jnp.float32,
    )
    h = jnp.maximum(h + b1_ref[...], 0.0).astype(jnp.bfloat16)
    out_ref[...] = (
        jnp.dot(h, w2_ref[...], preferred_element_type=jnp.float32) + b2_ref[...]
    )


def _gru_body(a0_ref, a1_ref, x_ref, wih_ref, whh_ref, bih_ref, bhh_ref, out_ref):
    agg = a0_ref[...] + a1_ref[...]
    x = x_ref[...]
    gi = jnp.dot(agg, wih_ref[...], preferred_element_type=jnp.float32) + bih_ref[...]
    gh = jnp.dot(x, whh_ref[...], preferred_element_type=jnp.float32) + bhh_ref[...]
    i_r = gi[:, :NODE_DIM]
    i_z = gi[:, NODE_DIM : 2 * NODE_DIM]
    i_n = gi[:, 2 * NODE_DIM :]
    h_r = gh[:, :NODE_DIM]
    h_z = gh[:, NODE_DIM : 2 * NODE_DIM]
    h_n = gh[:, 2 * NODE_DIM :]
    r = jax.nn.sigmoid(i_r + h_r)
    z = jax.nn.sigmoid(i_z + h_z)
    n = jnp.tanh(i_n + r * h_n)
    out_ref[...] = (1.0 - z) * n + z * x


BE = 6400
BN = 2000


def kernel(x, edge_index, edge_attr, W1, b1, W2, b2, W_ih, b_ih, W_hh, b_hh):
    src = edge_index[0].astype(jnp.int32)
    dst = edge_index[1].astype(jnp.int32)

    gathered = _make_sc_gather()(x, src)

    w1x_t = W1[:, :NODE_DIM].T.astype(jnp.bfloat16)
    w1e_t = W1[:, NODE_DIM:].T.astype(jnp.bfloat16)
    messages = pl.pallas_call(
        _mlp_body,
        grid=(N_EDGES // BE,),
        in_specs=[
            pl.BlockSpec((BE, NODE_DIM), lambda i: (i, 0)),
            pl.BlockSpec((EDGE_DIM, BE), lambda i: (0, i)),
            pl.BlockSpec((NODE_DIM, HIDDEN_DIM), lambda i: (0, 0)),
            pl.BlockSpec((EDGE_DIM, HIDDEN_DIM), lambda i: (0, 0)),
            pl.BlockSpec((1, HIDDEN_DIM), lambda i: (0, 0)),
            pl.BlockSpec((HIDDEN_DIM, HIDDEN_DIM), lambda i: (0, 0)),
            pl.BlockSpec((1, HIDDEN_DIM), lambda i: (0, 0)),
        ],
        out_specs=pl.BlockSpec((BE, HIDDEN_DIM), lambda i: (i, 0)),
        out_shape=jax.ShapeDtypeStruct((N_EDGES, HIDDEN_DIM), jnp.float32),
    )(
        gathered,
        edge_attr.T,
        w1x_t,
        w1e_t,
        b1.reshape(1, HIDDEN_DIM),
        W2.T.astype(jnp.bfloat16),
        b2.reshape(1, HIDDEN_DIM),
    )

    zero = jnp.zeros((N_NODES, HIDDEN_DIM), jnp.float32)
    agg2 = _make_sc_scatter()(messages, dst, zero)
    a0 = agg2[:N_NODES]
    a1 = agg2[N_NODES:]

    x_new = pl.pallas_call(
        _gru_body,
        grid=(N_NODES // BN,),
        in_specs=[
            pl.BlockSpec((BN, HIDDEN_DIM), lambda i: (i, 0)),
            pl.BlockSpec((BN, HIDDEN_DIM), lambda i: (i, 0)),
            pl.BlockSpec((BN, NODE_DIM), lambda i: (i, 0)),
            pl.BlockSpec((HIDDEN_DIM, 3 * NODE_DIM), lambda i: (0, 0)),
            pl.BlockSpec((NODE_DIM, 3 * NODE_DIM), lambda i: (0, 0)),
            pl.BlockSpec((1, 3 * NODE_DIM), lambda i: (0, 0)),
            pl.BlockSpec((1, 3 * NODE_DIM), lambda i: (0, 0)),
        ],
        out_specs=pl.BlockSpec((BN, NODE_DIM), lambda i: (i, 0)),
        out_shape=jax.ShapeDtypeStruct((N_NODES, NODE_DIM), jnp.float32),
    )(
        a0,
        a1,
        x,
        W_ih.T,
        W_hh.T,
        b_ih.reshape(1, 3 * NODE_DIM),
        b_hh.reshape(1, 3 * NODE_DIM),
    )
    return x_new
